# Initial kernel scaffold; baseline (speedup 1.0000x reference)
import jax
import jax.numpy as jnp
from jax.experimental import pallas as pl


def kernel(x, smooth, top_k_indices):
    raise NotImplementedError("write your pallas kernel here")


# speedup vs baseline: 6.9469x; 6.9469x over previous
import functools

import jax
import jax.numpy as jnp
from jax import lax
from jax.experimental import pallas as pl
from jax.experimental.pallas import tpu as pltpu
from jax.experimental.pallas import tpu_sc as plsc

FLAT = 2048
NIDX = 4096
LANES = 16


def _sc_build_table(smooth_f32, idx):
    mesh = plsc.VectorSubcoreMesh(core_axis_name="c", subcore_axis_name="s")

    @functools.partial(
        pl.kernel,
        mesh=mesh,
        out_type=jax.ShapeDtypeStruct((FLAT,), jnp.float32),
        scratch_types=[
            pltpu.VMEM((NIDX,), jnp.int32),
            pltpu.VMEM((NIDX,), jnp.float32),
            pltpu.VMEM((FLAT,), jnp.float32),
        ],
        compiler_params=pltpu.CompilerParams(needs_layout_passes=False),
    )
    def build(smooth_hbm, idx_hbm, out_hbm, idx_v, sm_v, m_v):
        cid = lax.axis_index("c")
        sid = lax.axis_index("s")

        @pl.when((cid == 0) & (sid == 0))
        def _():
            pltpu.sync_copy(idx_hbm, idx_v)
            pltpu.sync_copy(smooth_hbm, sm_v)

            ones = jnp.ones((LANES,), jnp.float32)

            def init_body(i, carry):
                m_v[pl.ds(i * LANES, LANES)] = ones
                return carry

            lax.fori_loop(0, FLAT // LANES, init_body, 0)

            def scat_body(j, carry):
                ii = idx_v[pl.ds(j * LANES, LANES)]
                ss = sm_v[pl.ds(j * LANES, LANES)]
                plsc.store_scatter(m_v, [ii], ss)
                return carry

            lax.fori_loop(0, NIDX // LANES, scat_body, 0)
            pltpu.sync_copy(m_v, out_hbm)

    return build(smooth_f32, idx)


def _tc_scale(x, m2):
    B, H, S, D = x.shape
    SB = 512

    def body(x_ref, m_ref, o_ref):
        h = pl.program_id(1)
        o_ref[...] = x_ref[...] * m_ref[pl.ds(h, 1), :]

    return pl.pallas_call(
        body,
        grid=(B, H, S // SB),
        in_specs=[
            pl.BlockSpec((1, 1, SB, D), lambda b, h, s: (b, h, s, 0)),
            pl.BlockSpec((H, D), lambda b, h, s: (0, 0)),
        ],
        out_specs=pl.BlockSpec((1, 1, SB, D), lambda b, h, s: (b, h, s, 0)),
        out_shape=jax.ShapeDtypeStruct(x.shape, x.dtype),
    )(x, m2)


def kernel(x, smooth, top_k_indices):
    m = _sc_build_table(smooth.astype(jnp.float32), top_k_indices)
    _, H, _, D = x.shape
    return _tc_scale(x, m.reshape(H, D))


# speedup vs baseline: 14.1850x; 2.0419x over previous
import functools

import jax
import jax.numpy as jnp
from jax import lax
from jax.experimental import pallas as pl
from jax.experimental.pallas import tpu as pltpu
from jax.experimental.pallas import tpu_sc as plsc

FLAT = 2048
NIDX = 4096
LANES = 16


def _sc_build_table(smooth_f32, idx):
    mesh = plsc.VectorSubcoreMesh(core_axis_name="c", subcore_axis_name="s")

    @functools.partial(
        pl.kernel,
        mesh=mesh,
        out_type=jax.ShapeDtypeStruct((FLAT,), jnp.float32),
        scratch_types=[
            pltpu.VMEM((NIDX,), jnp.int32),
            pltpu.VMEM((NIDX,), jnp.float32),
            pltpu.VMEM((FLAT,), jnp.float32),
        ],
        compiler_params=pltpu.CompilerParams(needs_layout_passes=False),
    )
    def build(smooth_hbm, idx_hbm, out_hbm, idx_v, sm_v, m_v):
        cid = lax.axis_index("c")
        sid = lax.axis_index("s")

        @pl.when((cid == 0) & (sid == 0))
        def _():
            pltpu.sync_copy(idx_hbm, idx_v)
            pltpu.sync_copy(smooth_hbm, sm_v)

            ones = jnp.ones((LANES,), jnp.float32)

            def init_body(i, carry):
                m_v[pl.ds(i * LANES, LANES)] = ones
                return carry

            lax.fori_loop(0, FLAT // LANES, init_body, 0)

            def scat_body(j, carry):
                ii = idx_v[pl.ds(j * LANES, LANES)]
                ss = sm_v[pl.ds(j * LANES, LANES)]
                plsc.store_scatter(m_v, [ii], ss)
                return carry

            lax.fori_loop(0, NIDX // LANES, scat_body, 0)
            pltpu.sync_copy(m_v, out_hbm)

    return build(smooth_f32, idx)


def _tc_scale(x, m2):
    B, H, S, D = x.shape
    SB = 2048

    def body(x_ref, m_ref, o_ref):
        h = pl.program_id(1)
        o_ref[...] = x_ref[...] * m_ref[pl.ds(h, 1), :]

    return pl.pallas_call(
        body,
        grid=(B, H, S // SB),
        in_specs=[
            pl.BlockSpec((1, 1, SB, D), lambda b, h, s: (b, h, s, 0)),
            pl.BlockSpec((H, D), lambda b, h, s: (0, 0)),
        ],
        out_specs=pl.BlockSpec((1, 1, SB, D), lambda b, h, s: (b, h, s, 0)),
        out_shape=jax.ShapeDtypeStruct(x.shape, x.dtype),
    )(x, m2)


def kernel(x, smooth, top_k_indices):
    m = _sc_build_table(smooth.astype(jnp.float32), top_k_indices)
    _, H, _, D = x.shape
    return _tc_scale(x, m.reshape(H, D))


# speedup vs baseline: 17.9156x; 1.2630x over previous
import functools

import jax
import jax.numpy as jnp
from jax import lax
from jax.experimental import pallas as pl
from jax.experimental.pallas import tpu as pltpu
from jax.experimental.pallas import tpu_sc as plsc

FLAT = 2048
NIDX = 4096
LANES = 16


def _sc_build_table(smooth_f32, idx):
    mesh = plsc.VectorSubcoreMesh(core_axis_name="c", subcore_axis_name="s")

    @functools.partial(
        pl.kernel,
        mesh=mesh,
        out_type=jax.ShapeDtypeStruct((FLAT,), jnp.float32),
        scratch_types=[
            pltpu.VMEM((NIDX,), jnp.int32),
            pltpu.VMEM((NIDX,), jnp.float32),
            pltpu.VMEM((FLAT,), jnp.float32),
        ],
        compiler_params=pltpu.CompilerParams(needs_layout_passes=False),
    )
    def build(smooth_hbm, idx_hbm, out_hbm, idx_v, sm_v, m_v):
        cid = lax.axis_index("c")
        sid = lax.axis_index("s")

        @pl.when((cid == 0) & (sid == 0))
        def _():
            pltpu.sync_copy(idx_hbm, idx_v)
            pltpu.sync_copy(smooth_hbm, sm_v)

            ones = jnp.ones((LANES,), jnp.float32)

            def init_body(i, carry):
                m_v[pl.ds(i * LANES, LANES)] = ones
                return carry

            lax.fori_loop(0, FLAT // LANES, init_body, 0)

            def scat_body(j, carry):
                ii = idx_v[pl.ds(j * LANES, LANES)]
                ss = sm_v[pl.ds(j * LANES, LANES)]
                plsc.store_scatter(m_v, [ii], ss)
                return carry

            lax.fori_loop(0, NIDX // LANES, scat_body, 0)
            pltpu.sync_copy(m_v, out_hbm)

    return build(smooth_f32, idx)


def _tc_scale(x, m2):
    B, H, S, D = x.shape
    SB = 4096

    def body(x_ref, m_ref, o_ref):
        h = pl.program_id(1)
        o_ref[...] = x_ref[...] * m_ref[pl.ds(h, 1), :]

    return pl.pallas_call(
        body,
        grid=(B, H, S // SB),
        in_specs=[
            pl.BlockSpec((1, 1, SB, D), lambda b, h, s: (b, h, s, 0)),
            pl.BlockSpec((H, D), lambda b, h, s: (0, 0)),
        ],
        out_specs=pl.BlockSpec((1, 1, SB, D), lambda b, h, s: (b, h, s, 0)),
        out_shape=jax.ShapeDtypeStruct(x.shape, x.dtype),
    )(x, m2)


def kernel(x, smooth, top_k_indices):
    m = _sc_build_table(smooth.astype(jnp.float32), top_k_indices)
    _, H, _, D = x.shape
    return _tc_scale(x, m.reshape(H, D))


# speedup vs baseline: 19.4539x; 1.0859x over previous
import functools

import jax
import jax.numpy as jnp
from jax import lax
from jax.experimental import pallas as pl
from jax.experimental.pallas import tpu as pltpu
from jax.experimental.pallas import tpu_sc as plsc

FLAT = 2048
NIDX = 4096
LANES = 16


def _sc_build_table(smooth_f32, idx):
    mesh = plsc.VectorSubcoreMesh(core_axis_name="c", subcore_axis_name="s")

    @functools.partial(
        pl.kernel,
        mesh=mesh,
        out_type=jax.ShapeDtypeStruct((FLAT,), jnp.float32),
        scratch_types=[
            pltpu.VMEM((NIDX,), jnp.int32),
            pltpu.VMEM((NIDX,), jnp.float32),
            pltpu.VMEM((FLAT,), jnp.float32),
        ],
        compiler_params=pltpu.CompilerParams(needs_layout_passes=False),
    )
    def build(smooth_hbm, idx_hbm, out_hbm, idx_v, sm_v, m_v):
        cid = lax.axis_index("c")
        sid = lax.axis_index("s")

        @pl.when((cid == 0) & (sid == 0))
        def _():
            pltpu.sync_copy(idx_hbm, idx_v)
            pltpu.sync_copy(smooth_hbm, sm_v)

            ones = jnp.ones((LANES,), jnp.float32)

            def init_body(i, carry):
                m_v[pl.ds(i * LANES, LANES)] = ones
                return carry

            lax.fori_loop(0, FLAT // LANES, init_body, 0)

            def scat_body(j, carry):
                ii = idx_v[pl.ds(j * LANES, LANES)]
                ss = sm_v[pl.ds(j * LANES, LANES)]
                plsc.store_scatter(m_v, [ii], ss)
                return carry

            lax.fori_loop(0, NIDX // LANES, scat_body, 0)
            pltpu.sync_copy(m_v, out_hbm)

    return build(smooth_f32, idx)


def _tc_scale(x, m2):
    B, H, S, D = x.shape
    HB = 4

    def body(x_ref, m_ref, o_ref):
        hb = pl.program_id(1)
        m = m_ref[pl.ds(hb * HB, HB), :].reshape(1, HB, 1, D)
        o_ref[...] = x_ref[...] * m

    return pl.pallas_call(
        body,
        grid=(B, H // HB),
        in_specs=[
            pl.BlockSpec((1, HB, S, D), lambda b, h: (b, h, 0, 0)),
            pl.BlockSpec((H, D), lambda b, h: (0, 0)),
        ],
        out_specs=pl.BlockSpec((1, HB, S, D), lambda b, h: (b, h, 0, 0)),
        out_shape=jax.ShapeDtypeStruct(x.shape, x.dtype),
    )(x, m2)


def kernel(x, smooth, top_k_indices):
    m = _sc_build_table(smooth.astype(jnp.float32), top_k_indices)
    _, H, _, D = x.shape
    return _tc_scale(x, m.reshape(H, D))


# speedup vs baseline: 19.7985x; 1.0177x over previous
import functools

import jax
import jax.numpy as jnp
from jax import lax
from jax.experimental import pallas as pl
from jax.experimental.pallas import tpu as pltpu
from jax.experimental.pallas import tpu_sc as plsc

FLAT = 2048
NIDX = 4096
LANES = 16


def _sc_build_table(smooth_f32, idx, ones):
    mesh = plsc.VectorSubcoreMesh(core_axis_name="c", subcore_axis_name="s")
    UNROLL = 4

    @functools.partial(
        pl.kernel,
        mesh=mesh,
        out_type=jax.ShapeDtypeStruct((FLAT,), jnp.float32),
        scratch_types=[
            pltpu.VMEM((NIDX,), jnp.int32),
            pltpu.VMEM((NIDX,), jnp.float32),
            pltpu.VMEM((FLAT,), jnp.float32),
            pltpu.SemaphoreType.DMA,
            pltpu.SemaphoreType.DMA,
            pltpu.SemaphoreType.DMA,
        ],
        compiler_params=pltpu.CompilerParams(needs_layout_passes=False),
    )
    def build(smooth_hbm, idx_hbm, ones_hbm, out_hbm, idx_v, sm_v, m_v,
              sem0, sem1, sem2):
        cid = lax.axis_index("c")
        sid = lax.axis_index("s")

        @pl.when((cid == 0) & (sid == 0))
        def _():
            c0 = pltpu.async_copy(idx_hbm, idx_v, sem0)
            c1 = pltpu.async_copy(smooth_hbm, sm_v, sem1)
            c2 = pltpu.async_copy(ones_hbm, m_v, sem2)
            c0.wait()
            c1.wait()
            c2.wait()

            def scat_body(j, carry):
                base = j * (LANES * UNROLL)
                for u in range(UNROLL):
                    ii = idx_v[pl.ds(base + u * LANES, LANES)]
                    ss = sm_v[pl.ds(base + u * LANES, LANES)]
                    plsc.store_scatter(m_v, [ii], ss)
                return carry

            lax.fori_loop(0, NIDX // (LANES * UNROLL), scat_body, 0)
            pltpu.sync_copy(m_v, out_hbm)

    return build(smooth_f32, idx, ones)


def _tc_scale(x, m2):
    B, H, S, D = x.shape
    HB = 4

    def body(x_ref, m_ref, o_ref):
        hb = pl.program_id(1)
        m = m_ref[pl.ds(hb * HB, HB), :].reshape(1, HB, 1, D)
        o_ref[...] = x_ref[...] * m

    return pl.pallas_call(
        body,
        grid=(B, H // HB),
        in_specs=[
            pl.BlockSpec((1, HB, S, D), lambda b, h: (b, h, 0, 0)),
            pl.BlockSpec((H, D), lambda b, h: (0, 0)),
        ],
        out_specs=pl.BlockSpec((1, HB, S, D), lambda b, h: (b, h, 0, 0)),
        out_shape=jax.ShapeDtypeStruct(x.shape, x.dtype),
    )(x, m2)


def kernel(x, smooth, top_k_indices):
    ones = jnp.ones((FLAT,), jnp.float32)
    m = _sc_build_table(smooth.astype(jnp.float32), top_k_indices, ones)
    _, H, _, D = x.shape
    return _tc_scale(x, m.reshape(H, D))


# speedup vs baseline: 19.8882x; 1.0045x over previous
import functools

import jax
import jax.numpy as jnp
from jax import lax
from jax.experimental import pallas as pl
from jax.experimental.pallas import tpu as pltpu
from jax.experimental.pallas import tpu_sc as plsc

FLAT = 2048
NIDX = 4096
LANES = 16


def _sc_build_table(smooth_f32, idx, ones):
    mesh = plsc.VectorSubcoreMesh(core_axis_name="c", subcore_axis_name="s")
    UNROLL = 8

    @functools.partial(
        pl.kernel,
        mesh=mesh,
        out_type=jax.ShapeDtypeStruct((FLAT,), jnp.float32),
        scratch_types=[
            pltpu.VMEM((NIDX,), jnp.int32),
            pltpu.VMEM((NIDX,), jnp.float32),
            pltpu.VMEM((FLAT,), jnp.float32),
            pltpu.SemaphoreType.DMA,
            pltpu.SemaphoreType.DMA,
            pltpu.SemaphoreType.DMA,
        ],
        compiler_params=pltpu.CompilerParams(needs_layout_passes=False),
    )
    def build(smooth_hbm, idx_hbm, ones_hbm, out_hbm, idx_v, sm_v, m_v,
              sem0, sem1, sem2):
        cid = lax.axis_index("c")
        sid = lax.axis_index("s")

        @pl.when((cid == 0) & (sid == 0))
        def _():
            c0 = pltpu.async_copy(idx_hbm, idx_v, sem0)
            c1 = pltpu.async_copy(smooth_hbm, sm_v, sem1)
            c2 = pltpu.async_copy(ones_hbm, m_v, sem2)
            c0.wait()
            c1.wait()
            c2.wait()

            def scat_body(j, carry):
                base = j * (LANES * UNROLL)
                for u in range(UNROLL):
                    ii = idx_v[pl.ds(base + u * LANES, LANES)]
                    ss = sm_v[pl.ds(base + u * LANES, LANES)]
                    plsc.store_scatter(m_v, [ii], ss)
                return carry

            lax.fori_loop(0, NIDX // (LANES * UNROLL), scat_body, 0)
            pltpu.sync_copy(m_v, out_hbm)

    return build(smooth_f32, idx, ones)


def _tc_scale(x, m2):
    B, H, S, D = x.shape
    HB = 4

    def body(x_ref, m_ref, o_ref):
        hb = pl.program_id(1)
        m = m_ref[pl.ds(hb * HB, HB), :].reshape(1, HB, 1, D)
        o_ref[...] = x_ref[...] * m

    return pl.pallas_call(
        body,
        grid=(B, H // HB),
        in_specs=[
            pl.BlockSpec((1, HB, S, D), lambda b, h: (b, h, 0, 0)),
            pl.BlockSpec((H, D), lambda b, h: (0, 0)),
        ],
        out_specs=pl.BlockSpec((1, HB, S, D), lambda b, h: (b, h, 0, 0)),
        out_shape=jax.ShapeDtypeStruct(x.shape, x.dtype),
    )(x, m2)


def kernel(x, smooth, top_k_indices):
    ones = jnp.ones((FLAT,), jnp.float32)
    m = _sc_build_table(smooth.astype(jnp.float32), top_k_indices, ones)
    _, H, _, D = x.shape
    return _tc_scale(x, m.reshape(H, D))
